# baseline (speedup 1.0000x reference)
import functools

import jax
import jax.numpy as jnp
from jax.experimental import pallas as pl
from jax.experimental.pallas import tpu as pltpu
from jax.experimental.pallas import tpu_sc as plsc

K_NEIGHBORS = 10
N, KSHOT, Q = 100, 5, 15
D_IN, D_OUT = 2048, 1024
NQ = N * Q
NQP = 1536
NP = 128
ROWS = N * (KSHOT + Q)
BLK = 768
NB = NQP // BLK
NEG = -1e30
BIG = 1e30

NWORKER = 32
RPW = NQP // NWORKER
NCH = NQP // 16


def _enc_kernel(x_ref, w_ref, b_ref, out_ref):
    out_ref[:] = (
        jnp.dot(x_ref[:], w_ref[:], preferred_element_type=jnp.float32) + b_ref[:]
    )


def _p1a_kernel(sup_ref, q_ref, sim_out, apn_out, qn_out, qn_scr):
    s = pl.program_id(0)

    @pl.when(s == 0)
    def _phase_proto():
        sup = sup_ref[:]
        qf = q_ref[:]
        proto = (
            sup[0 * NP:1 * NP] + sup[1 * NP:2 * NP] + sup[2 * NP:3 * NP]
            + sup[3 * NP:4 * NP] + sup[4 * NP:5 * NP]
        ) / float(KSHOT)
        pnorm = jnp.sqrt(jnp.sum(proto * proto, axis=1, keepdims=True))
        pn = proto / (pnorm + 1e-8)
        qnorm = jnp.sqrt(jnp.sum(qf * qf, axis=1, keepdims=True))
        qn = qf / (qnorm + 1e-8)
        qn_scr[:] = qn
        qn_out[:] = qn

        pre = jax.lax.dot_general(
            qn, pn, (((1,), (1,)), ((), ())), preferred_element_type=jnp.float32
        )
        colid = jax.lax.broadcasted_iota(jnp.int32, (NQP, NP), 1)
        rowid = jax.lax.broadcasted_iota(jnp.int32, (NQP, NP), 0)
        pre_m = jnp.where(colid < N, pre, NEG)
        rowmax = jnp.max(pre_m, axis=1, keepdims=True)
        idx = jnp.where(pre_m == rowmax, colid, jnp.int32(2**30))
        amin = jnp.min(idx, axis=1, keepdims=True)
        onehot = (colid == amin) & (rowid < NQ)
        exp_ref_w = jnp.where(onehot, jnp.exp(pre), 0.0)

        self_sim = jnp.sum(pn * pn, axis=1, keepdims=True)
        exp_self = jnp.exp(self_sim)
        ones = jnp.ones((NQP, 1), dtype=jnp.float32)
        denom = jax.lax.dot_general(
            exp_ref_w, ones, (((0,), (0,)), ((), ())),
            preferred_element_type=jnp.float32,
        ) + exp_self
        num = jax.lax.dot_general(
            exp_ref_w, qf, (((0,), (0,)), ((), ())),
            preferred_element_type=jnp.float32,
        ) + exp_self * proto
        ap = num / denom
        apnorm = jnp.sqrt(jnp.sum(ap * ap, axis=1, keepdims=True))
        apn_out[:] = ap / (apnorm + 1e-8)

    @pl.when(s == 1)
    def _phase_sim():
        a = qn_scr[pl.ds(0, BLK), :]
        b = qn_scr[:]
        sim_out[:] = jax.lax.dot_general(
            a, b, (((1,), (1,)), ((), ())), preferred_element_type=jnp.float32
        )


def _p1b_kernel(qn_ref, sim_out):
    a = qn_ref[pl.ds(BLK, BLK), :]
    b = qn_ref[:]
    sim_out[:] = jax.lax.dot_general(
        a, b, (((1,), (1,)), ((), ())), preferred_element_type=jnp.float32
    )


def _gather16(x, idx):
    return jax.lax.gather(
        x, idx.reshape(16, 1),
        jax.lax.GatherDimensionNumbers(
            offset_dims=(), collapsed_slice_dims=(0,), start_index_map=(0,)
        ),
        (1,), mode=jax.lax.GatherScatterMode.PROMISE_IN_BOUNDS,
    )


def _xlane_max(m):
    lane = jax.lax.iota(jnp.int32, 16)
    for sh in (1, 2, 4, 8):
        m = jnp.maximum(m, _gather16(m, jnp.bitwise_xor(lane, sh)))
    return m


HH = NQP // 2
RPWH = HH // NWORKER


def _make_sc_thr(row0):
    @functools.partial(
        pl.kernel,
        mesh=plsc.VectorSubcoreMesh(core_axis_name="c", subcore_axis_name="s"),
        out_type=jax.ShapeDtypeStruct((NWORKER, 32), jnp.float32),
        scratch_types=[
            pltpu.VMEM((32, NQP), jnp.float32),
            pltpu.VMEM((32,), jnp.float32),
        ],
    )
    def _sc_thr_kernel(sim_hbm, thr_hbm, blk_v, thr_v):
        wid = jax.lax.axis_index("s") * 2 + jax.lax.axis_index("c")
        base = wid * RPWH
        pltpu.sync_copy(sim_hbm.at[pl.ds(base, RPWH)], blk_v.at[pl.ds(0, RPWH)])

        lane = jax.lax.iota(jnp.int32, 16)
        ILV = 2

        def group_body(g, carry):
            tvec = jnp.full((16,), BIG, dtype=jnp.float32)
            for qd in range(16 // ILV):
                rows = [g * 16 + qd * ILV + u for u in range(ILV)]

                def chunk_body(c, stacks):
                    out = []
                    for u in range(ILV):
                        v = blk_v[rows[u], pl.ds(c * 16, 16)]
                        v = jnp.where(c * 16 + lane < NQ, v, NEG)
                        stack = list(stacks[u])
                        for t in range(K_NEIGHBORS):
                            hi = jnp.maximum(stack[t], v)
                            v = jnp.minimum(stack[t], v)
                            stack[t] = hi
                        out.append(tuple(stack))
                    return tuple(out)

                stacks = tuple(
                    tuple(jnp.full((16,), NEG, dtype=jnp.float32)
                          for _ in range(K_NEIGHBORS))
                    for _ in range(ILV)
                )
                stacks = jax.lax.fori_loop(0, NCH, chunk_body, stacks)

                for u in range(ILV):
                    stack = list(stacks[u])
                    thr = jnp.full((16,), NEG, dtype=jnp.float32)
                    for _ in range(K_NEIGHBORS):
                        m16 = stack[0]
                        for t in range(1, K_NEIGHBORS):
                            m16 = jnp.maximum(m16, stack[t])
                        thr = _xlane_max(m16)
                        for t in range(K_NEIGHBORS):
                            stack[t] = jnp.where(stack[t] == thr, NEG, stack[t])
                    thr = jnp.where(
                        row0 + base + rows[u] < NQ, thr,
                        jnp.full((16,), BIG, dtype=jnp.float32),
                    )
                    tvec = jnp.where(lane == qd * ILV + u, thr, tvec)
            thr_v[pl.ds(g * 16, 16)] = tvec
            return carry

        jax.lax.fori_loop(0, 2, group_body, jnp.int32(0))
        pltpu.sync_copy(thr_v, thr_hbm.at[wid])

    return _sc_thr_kernel


_sc_thr_top = _make_sc_thr(0)
_sc_thr_bot = _make_sc_thr(HH)


def _p2_kernel(sim_t_ref, sim_b_ref, ti_ref, tj_ref, q_ref, apn_ref, tao_ref,
               out_ref):
    s = pl.program_id(0)

    def agg(sim):
        ti = ti_ref[:]
        tj = tj_ref[:]
        w = jnp.where((sim >= ti) & (sim >= tj), jnp.exp(sim), 0.0)
        ssum = jnp.sum(w, axis=1, keepdims=True)
        ssum = jnp.where(ssum > 0.0, ssum, 1.0)
        aq = jnp.dot(w, q_ref[:], preferred_element_type=jnp.float32) / ssum
        anorm = jnp.sqrt(jnp.sum(aq * aq, axis=1, keepdims=True))
        aqn = aq / (anorm + 1e-8)
        out_ref[:] = tao_ref[0, 0] * jax.lax.dot_general(
            aqn, apn_ref[:], (((1,), (1,)), ((), ())),
            preferred_element_type=jnp.float32,
        )

    @pl.when(s == 0)
    def _top():
        agg(sim_t_ref[:])

    @pl.when(s == 1)
    def _bot():
        agg(sim_b_ref[:])


def kernel(x, W_enc, b_enc, tao, n, k, q):
    f32 = jnp.float32

    feat = pl.pallas_call(
        _enc_kernel,
        grid=(4,),
        in_specs=[
            pl.BlockSpec((512, D_IN), lambda i: (i, 0)),
            pl.BlockSpec((D_IN, D_OUT), lambda i: (0, 0)),
            pl.BlockSpec((1, D_OUT), lambda i: (0, 0)),
        ],
        out_specs=pl.BlockSpec((512, D_OUT), lambda i: (i, 0)),
        out_shape=jax.ShapeDtypeStruct((ROWS, D_OUT), f32),
    )(x, W_enc, b_enc.reshape(1, D_OUT))

    f3 = feat.reshape(N, KSHOT + Q, D_OUT)
    sup3 = jnp.pad(f3[:, :KSHOT], ((0, NP - N), (0, 0), (0, 0)))
    sup = jnp.transpose(sup3, (1, 0, 2)).reshape(KSHOT * NP, D_OUT)
    qf = f3[:, KSHOT:].reshape(NQ, D_OUT)
    qf = jnp.pad(qf, ((0, NQP - NQ), (0, 0)))

    sim_top, apn, qn = pl.pallas_call(
        _p1a_kernel,
        grid=(2,),
        in_specs=[
            pl.BlockSpec((KSHOT * NP, D_OUT), lambda i: (0, 0)),
            pl.BlockSpec((NQP, D_OUT), lambda i: (0, 0)),
        ],
        out_specs=[
            pl.BlockSpec((BLK, NQP), lambda i: (0, 0)),
            pl.BlockSpec((NP, D_OUT), lambda i: (0, 0)),
            pl.BlockSpec((NQP, D_OUT), lambda i: (0, 0)),
        ],
        out_shape=[
            jax.ShapeDtypeStruct((HH, NQP), f32),
            jax.ShapeDtypeStruct((NP, D_OUT), f32),
            jax.ShapeDtypeStruct((NQP, D_OUT), f32),
        ],
        scratch_shapes=[pltpu.VMEM((NQP, D_OUT), f32)],
    )(sup, qf)

    thr2_top = _sc_thr_top(sim_top)

    sim_bot = pl.pallas_call(
        _p1b_kernel,
        in_specs=[pl.BlockSpec((NQP, D_OUT), lambda: (0, 0))],
        out_specs=pl.BlockSpec((BLK, NQP), lambda: (0, 0)),
        out_shape=jax.ShapeDtypeStruct((HH, NQP), f32),
    )(qn)

    thr2_bot = _sc_thr_bot(sim_bot)

    thr = jnp.concatenate(
        [thr2_top[:, :RPWH].reshape(HH), thr2_bot[:, :RPWH].reshape(HH)]
    )

    out = pl.pallas_call(
        _p2_kernel,
        grid=(NB,),
        in_specs=[
            pl.BlockSpec((BLK, NQP), lambda i: (0, 0)),
            pl.BlockSpec((BLK, NQP), lambda i: (0, 0)),
            pl.BlockSpec((BLK, 1), lambda i: (i, 0)),
            pl.BlockSpec((1, NQP), lambda i: (0, 0)),
            pl.BlockSpec((NQP, D_OUT), lambda i: (0, 0)),
            pl.BlockSpec((NP, D_OUT), lambda i: (0, 0)),
            pl.BlockSpec((1, 1), lambda i: (0, 0)),
        ],
        out_specs=pl.BlockSpec((BLK, NP), lambda i: (i, 0)),
        out_shape=jax.ShapeDtypeStruct((NQP, NP), f32),
    )(sim_top, sim_bot, thr.reshape(NQP, 1), thr.reshape(1, NQP), qf, apn,
      tao.reshape(1, 1))

    return out[:NQ, :N]


# speedup vs baseline: 1.1820x; 1.1820x over previous
import functools

import jax
import jax.numpy as jnp
from jax.experimental import pallas as pl
from jax.experimental.pallas import tpu as pltpu
from jax.experimental.pallas import tpu_sc as plsc

K_NEIGHBORS = 10
N, KSHOT, Q = 100, 5, 15
D_IN, D_OUT = 2048, 1024
NQ = N * Q
NQP = 1536
NP = 128
ROWS = N * (KSHOT + Q)
BLK = 768
NB = NQP // BLK
NEG = -1e30
BIG = 1e30

NWORKER = 32
RPW = NQP // NWORKER
NCH = NQP // 16


def _enc_kernel(x_ref, w_ref, b_ref, out_ref):
    out_ref[:] = (
        jnp.dot(x_ref[:], w_ref[:], preferred_element_type=jnp.float32) + b_ref[:]
    )


def _p1_kernel(sup_ref, q_ref, sim_out, apn_out, qn_scr):
    s = pl.program_id(0)

    @pl.when(s == 0)
    def _phase_proto():
        sup = sup_ref[:]
        qf = q_ref[:]
        proto = (
            sup[0 * NP:1 * NP] + sup[1 * NP:2 * NP] + sup[2 * NP:3 * NP]
            + sup[3 * NP:4 * NP] + sup[4 * NP:5 * NP]
        ) / float(KSHOT)
        pnorm = jnp.sqrt(jnp.sum(proto * proto, axis=1, keepdims=True))
        pn = proto / (pnorm + 1e-8)
        qnorm = jnp.sqrt(jnp.sum(qf * qf, axis=1, keepdims=True))
        qn = qf / (qnorm + 1e-8)
        qn_scr[:] = qn

        pre = jax.lax.dot_general(
            qn, pn, (((1,), (1,)), ((), ())), preferred_element_type=jnp.float32
        )
        colid = jax.lax.broadcasted_iota(jnp.int32, (NQP, NP), 1)
        rowid = jax.lax.broadcasted_iota(jnp.int32, (NQP, NP), 0)
        pre_m = jnp.where(colid < N, pre, NEG)
        rowmax = jnp.max(pre_m, axis=1, keepdims=True)
        idx = jnp.where(pre_m == rowmax, colid, jnp.int32(2**30))
        amin = jnp.min(idx, axis=1, keepdims=True)
        onehot = (colid == amin) & (rowid < NQ)
        exp_ref_w = jnp.where(onehot, jnp.exp(pre), 0.0)

        self_sim = jnp.sum(pn * pn, axis=1, keepdims=True)
        exp_self = jnp.exp(self_sim)
        ones = jnp.ones((NQP, 1), dtype=jnp.float32)
        denom = jax.lax.dot_general(
            exp_ref_w, ones, (((0,), (0,)), ((), ())),
            preferred_element_type=jnp.float32,
        ) + exp_self
        num = jax.lax.dot_general(
            exp_ref_w, qf, (((0,), (0,)), ((), ())),
            preferred_element_type=jnp.float32,
        ) + exp_self * proto
        ap = num / denom
        apnorm = jnp.sqrt(jnp.sum(ap * ap, axis=1, keepdims=True))
        apn_out[:] = ap / (apnorm + 1e-8)

    @pl.when(s >= 1)
    def _phase_sim():
        t = s - 1
        a = qn_scr[pl.ds(t * BLK, BLK), :]
        b = qn_scr[:]
        sim_out[:] = jax.lax.dot_general(
            a, b, (((1,), (1,)), ((), ())), preferred_element_type=jnp.float32
        )


def _gather16(x, idx):
    return jax.lax.gather(
        x, idx.reshape(16, 1),
        jax.lax.GatherDimensionNumbers(
            offset_dims=(), collapsed_slice_dims=(0,), start_index_map=(0,)
        ),
        (1,), mode=jax.lax.GatherScatterMode.PROMISE_IN_BOUNDS,
    )


def _xlane_max(m):
    lane = jax.lax.iota(jnp.int32, 16)
    for sh in (1, 2, 4, 8):
        m = jnp.maximum(m, _gather16(m, jnp.bitwise_xor(lane, sh)))
    return m


@functools.partial(
    pl.kernel,
    mesh=plsc.VectorSubcoreMesh(core_axis_name="c", subcore_axis_name="s"),
    out_type=jax.ShapeDtypeStruct((NQP,), jnp.float32),
    scratch_types=[
        pltpu.VMEM((RPW, NQP), jnp.float32),
        pltpu.VMEM((RPW,), jnp.float32),
    ],
)
def _sc_thr_kernel(sim_hbm, thr_hbm, blk_v, thr_v):
    wid = jax.lax.axis_index("s") * 2 + jax.lax.axis_index("c")
    base = wid * RPW
    pltpu.sync_copy(sim_hbm.at[pl.ds(base, RPW)], blk_v)

    lane = jax.lax.iota(jnp.int32, 16)
    ILV = 2

    def group_body(g, carry):
        tvec = jnp.full((16,), BIG, dtype=jnp.float32)
        for qd in range(16 // ILV):
            rows = [g * 16 + qd * ILV + u for u in range(ILV)]

            def chunk_body(c, stacks):
                out = []
                for u in range(ILV):
                    v = blk_v[rows[u], pl.ds(c * 16, 16)]
                    v = jnp.where(c * 16 + lane < NQ, v, NEG)
                    stack = list(stacks[u])
                    for t in range(K_NEIGHBORS):
                        hi = jnp.maximum(stack[t], v)
                        v = jnp.minimum(stack[t], v)
                        stack[t] = hi
                    out.append(tuple(stack))
                return tuple(out)

            stacks = tuple(
                tuple(jnp.full((16,), NEG, dtype=jnp.float32)
                      for _ in range(K_NEIGHBORS))
                for _ in range(ILV)
            )
            stacks = jax.lax.fori_loop(0, NCH, chunk_body, stacks)

            for u in range(ILV):
                stack = list(stacks[u])
                thr = jnp.full((16,), NEG, dtype=jnp.float32)
                for _ in range(K_NEIGHBORS):
                    m16 = stack[0]
                    for t in range(1, K_NEIGHBORS):
                        m16 = jnp.maximum(m16, stack[t])
                    thr = _xlane_max(m16)
                    for t in range(K_NEIGHBORS):
                        stack[t] = jnp.where(stack[t] == thr, NEG, stack[t])
                thr = jnp.where(
                    base + rows[u] < NQ, thr,
                    jnp.full((16,), BIG, dtype=jnp.float32),
                )
                tvec = jnp.where(lane == qd * ILV + u, thr, tvec)
        thr_v[pl.ds(g * 16, 16)] = tvec
        return carry

    jax.lax.fori_loop(0, RPW // 16, group_body, jnp.int32(0))
    pltpu.sync_copy(thr_v, thr_hbm.at[pl.ds(base, RPW)])


def _p2_kernel(sim_ref, ti_ref, tj_ref, q_ref, apn_ref, tao_ref, out_ref):
    sim = sim_ref[:]
    ti = ti_ref[:]
    tj = tj_ref[:]
    w = jnp.where((sim >= ti) & (sim >= tj), jnp.exp(sim), 0.0)
    ssum = jnp.sum(w, axis=1, keepdims=True)
    ssum = jnp.where(ssum > 0.0, ssum, 1.0)
    aq = jnp.dot(w, q_ref[:], preferred_element_type=jnp.float32) / ssum
    anorm = jnp.sqrt(jnp.sum(aq * aq, axis=1, keepdims=True))
    aqn = aq / (anorm + 1e-8)
    out_ref[:] = tao_ref[0, 0] * jax.lax.dot_general(
        aqn, apn_ref[:], (((1,), (1,)), ((), ())),
        preferred_element_type=jnp.float32,
    )


def kernel(x, W_enc, b_enc, tao, n, k, q):
    f32 = jnp.float32

    feat = pl.pallas_call(
        _enc_kernel,
        grid=(4,),
        in_specs=[
            pl.BlockSpec((512, D_IN), lambda i: (i, 0)),
            pl.BlockSpec((D_IN, D_OUT), lambda i: (0, 0)),
            pl.BlockSpec((1, D_OUT), lambda i: (0, 0)),
        ],
        out_specs=pl.BlockSpec((512, D_OUT), lambda i: (i, 0)),
        out_shape=jax.ShapeDtypeStruct((ROWS, D_OUT), f32),
    )(x, W_enc, b_enc.reshape(1, D_OUT))

    f3 = feat.reshape(N, KSHOT + Q, D_OUT)
    sup3 = jnp.pad(f3[:, :KSHOT], ((0, NP - N), (0, 0), (0, 0)))
    sup = jnp.transpose(sup3, (1, 0, 2)).reshape(KSHOT * NP, D_OUT)
    qf = f3[:, KSHOT:].reshape(NQ, D_OUT)
    qf = jnp.pad(qf, ((0, NQP - NQ), (0, 0)))

    sim, apn = pl.pallas_call(
        _p1_kernel,
        grid=(NB + 1,),
        in_specs=[
            pl.BlockSpec((KSHOT * NP, D_OUT), lambda i: (0, 0)),
            pl.BlockSpec((NQP, D_OUT), lambda i: (0, 0)),
        ],
        out_specs=[
            pl.BlockSpec((BLK, NQP), lambda i: (jnp.clip(i - 1, 0, NB - 1), 0)),
            pl.BlockSpec((NP, D_OUT), lambda i: (0, 0)),
        ],
        out_shape=[
            jax.ShapeDtypeStruct((NQP, NQP), f32),
            jax.ShapeDtypeStruct((NP, D_OUT), f32),
        ],
        scratch_shapes=[pltpu.VMEM((NQP, D_OUT), f32)],
    )(sup, qf)

    thr = _sc_thr_kernel(sim)

    out = pl.pallas_call(
        _p2_kernel,
        grid=(NB,),
        in_specs=[
            pl.BlockSpec((BLK, NQP), lambda i: (i, 0)),
            pl.BlockSpec((BLK, 1), lambda i: (i, 0)),
            pl.BlockSpec((1, NQP), lambda i: (0, 0)),
            pl.BlockSpec((NQP, D_OUT), lambda i: (0, 0)),
            pl.BlockSpec((NP, D_OUT), lambda i: (0, 0)),
            pl.BlockSpec((1, 1), lambda i: (0, 0)),
        ],
        out_specs=pl.BlockSpec((BLK, NP), lambda i: (i, 0)),
        out_shape=jax.ShapeDtypeStruct((NQP, NP), f32),
    )(sim, thr.reshape(NQP, 1), thr.reshape(1, NQP), qf, apn, tao.reshape(1, 1))

    return out[:NQ, :N]
